# baseline (speedup 1.0000x reference)
import functools

import jax
import jax.numpy as jnp
from jax import lax
from jax.experimental import pallas as pl
from jax.experimental.pallas import tpu as pltpu
from jax.experimental.pallas import tpu_sc as plsc

H = 128
ATOM_F = 128
BOND_F = 16
N_MOLS = 200
ATOMS_PER_MOL = 50
N_ATOMS = 1 + N_MOLS * ATOMS_PER_MOL
N_BONDS = 1 + 320000
MAX_NB = 32

NW = 32
A_CHUNK = 4
PER_A = 320
ATOM_PAD = NW * PER_A
B_CHUNK = 128
PER_B = 10112
BOND_PAD = NW * PER_B


def _worker_id():
    return lax.axis_index("s") * 2 + lax.axis_index("c")


_G = A_CHUNK * MAX_NB
_NCH_A = PER_A // A_CHUNK


def _atom_agg_body(mb_hbm, a2b_hbm, ma_hbm, out_hbm,
                   idx_all, ma_all, rows_v, g0, g1, g2, g3):
    wid = _worker_id()
    ibase = pl.multiple_of(wid * (PER_A * MAX_NB), 1024)
    abase = pl.multiple_of(wid * PER_A, 64)
    pltpu.sync_copy(a2b_hbm.at[pl.ds(ibase, PER_A * MAX_NB)], idx_all)
    pltpu.sync_copy(ma_hbm.at[pl.ds(abase, PER_A)], ma_all)

    def issue(c, half, sem):
        ioff = pl.multiple_of(c * _G, _G)
        pltpu.async_copy(mb_hbm.at[idx_all.at[pl.ds(ioff, _G)]],
                         rows_v.at[pl.ds(half * _G, _G)], sem)

    gsems = (g0, g1, g2, g3)
    for p in range(3):
        issue(p, p, gsems[p])

    def chunk_body(ci, _):
        par = lax.rem(ci, 4)
        nxt = ci + 3

        for p in range(4):
            @pl.when(jnp.logical_and(nxt < _NCH_A, par == p))
            def _(p=p):
                issue(nxt, (p + 3) % 4, gsems[(p + 3) % 4])

        for p in range(4):
            @pl.when(par == p)
            def _(p=p):
                pltpu.make_async_copy(
                    mb_hbm.at[pl.ds(0, _G)],
                    rows_v.at[pl.ds(p * _G, _G)], gsems[p]).wait()

        off = par * _G
        aoff = ci * A_CHUNK
        for a in range(A_CHUNK):
            for c8 in range(H // 16):
                sl = pl.ds(c8 * 16, 16)
                vals = [rows_v[off + a * MAX_NB + j, sl] for j in range(MAX_NB)]
                m = vals[0]
                for j in range(1, MAX_NB):
                    m = jnp.maximum(m, vals[j])
                b = [((vals[s] + vals[s + 8]) + vals[s + 16]) + vals[s + 24]
                     for s in range(8)]
                c = [b[s] + b[s + 4] for s in range(4)]
                e = (c[0] + c[2]) + (c[1] + c[3])
                ma_all[aoff + a, sl] = ma_all[aoff + a, sl] + e * m
        return 0

    lax.fori_loop(0, _NCH_A, chunk_body, 0)
    pltpu.sync_copy(ma_all, out_hbm.at[pl.ds(abase, PER_A)])


_NCH_B = PER_B // B_CHUNK


def _bond_gather_body(ma_hbm, mb_hbm, b2a_hbm, b2revb_hbm, out_hbm,
                      idxa_all, idxr_all, bufa, bufr,
                      ga0, ga1, ga2, gr0, gr1, gr2, o0, o1, o2):
    wid = _worker_id()
    bbase = pl.multiple_of(wid * PER_B, 128)
    pltpu.sync_copy(b2a_hbm.at[pl.ds(bbase, PER_B)], idxa_all)
    pltpu.sync_copy(b2revb_hbm.at[pl.ds(bbase, PER_B)], idxr_all)

    gas = (ga0, ga1, ga2)
    grs = (gr0, gr1, gr2)
    os_ = (o0, o1, o2)

    def issue(c, half):
        coff = pl.multiple_of(c * B_CHUNK, B_CHUNK)
        pltpu.async_copy(ma_hbm.at[idxa_all.at[pl.ds(coff, B_CHUNK)]],
                         bufa.at[pl.ds(half * B_CHUNK, B_CHUNK)], gas[half])
        pltpu.async_copy(mb_hbm.at[idxr_all.at[pl.ds(coff, B_CHUNK)]],
                         bufr.at[pl.ds(half * B_CHUNK, B_CHUNK)], grs[half])

    def wait_g(half):
        sl = pl.ds(half * B_CHUNK, B_CHUNK)
        pltpu.make_async_copy(ma_hbm.at[pl.ds(0, B_CHUNK)], bufa.at[sl],
                              gas[half]).wait()
        pltpu.make_async_copy(mb_hbm.at[pl.ds(0, B_CHUNK)], bufr.at[sl],
                              grs[half]).wait()

    def wait_o(half):
        pltpu.make_async_copy(bufa.at[pl.ds(half * B_CHUNK, B_CHUNK)],
                              out_hbm.at[pl.ds(0, B_CHUNK)], os_[half]).wait()

    issue(0, 0)
    issue(1, 1)

    def chunk_body(ci, _):
        par = lax.rem(ci, 3)
        nxt = ci + 2

        for p in range(3):
            q = (p + 2) % 3
            cond = jnp.logical_and(nxt < _NCH_B, par == p)

            @pl.when(jnp.logical_and(cond, ci >= 1))
            def _(q=q):
                wait_o(q)

            @pl.when(cond)
            def _(q=q):
                issue(nxt, q)

        for p in range(3):
            @pl.when(par == p)
            def _(p=p):
                wait_g(p)

        off = par * B_CHUNK

        def row_body(rr, _):
            for u in range(4):
                r = off + rr * 4 + u
                for c8 in range(H // 16):
                    sl = pl.ds(c8 * 16, 16)
                    bufa[r, sl] = bufa[r, sl] - bufr[r, sl]
            return 0

        lax.fori_loop(0, B_CHUNK // 4, row_body, 0)

        base = pl.multiple_of(wid * PER_B + ci * B_CHUNK, B_CHUNK)

        for p in range(3):
            @pl.when(par == p)
            def _(p=p):
                pltpu.async_copy(bufa.at[pl.ds(p * B_CHUNK, B_CHUNK)],
                                 out_hbm.at[pl.ds(base, B_CHUNK)], os_[p])

        return 0

    lax.fori_loop(0, _NCH_B, chunk_body, 0)
    wait_o(0)
    wait_o(1)
    wait_o(2)


@functools.lru_cache(maxsize=None)
def _sc_kernels():
    mesh = plsc.VectorSubcoreMesh(core_axis_name="c", subcore_axis_name="s")
    atom_agg = pl.kernel(
        _atom_agg_body,
        out_type=jax.ShapeDtypeStruct((ATOM_PAD, H), jnp.float32),
        mesh=mesh,
        scratch_types=[
            pltpu.VMEM((PER_A * MAX_NB,), jnp.int32),
            pltpu.VMEM((PER_A, H), jnp.float32),
            pltpu.VMEM((4 * _G, H), jnp.float32),
        ] + [pltpu.SemaphoreType.DMA] * 4,
    )
    bond_gather = pl.kernel(
        _bond_gather_body,
        out_type=jax.ShapeDtypeStruct((BOND_PAD, H), jnp.float32),
        mesh=mesh,
        scratch_types=[
            pltpu.VMEM((PER_B,), jnp.int32),
            pltpu.VMEM((PER_B,), jnp.int32),
            pltpu.VMEM((3 * B_CHUNK, H), jnp.float32),
            pltpu.VMEM((3 * B_CHUNK, H), jnp.float32),
        ] + [pltpu.SemaphoreType.DMA] * 9,
    )
    return atom_agg, bond_gather


def _mm_relu_body(x_ref, w_ref, o_ref):
    o_ref[...] = jnp.maximum(
        jnp.dot(x_ref[...], w_ref[...], preferred_element_type=jnp.float32), 0.0)


def _mm_relu(x, wT, chunk):
    R, K = x.shape
    N = wT.shape[1]
    return pl.pallas_call(
        _mm_relu_body,
        grid=(R // chunk,),
        in_specs=[
            pl.BlockSpec((chunk, K), lambda i: (i, 0)),
            pl.BlockSpec((K, N), lambda i: (0, 0)),
        ],
        out_specs=pl.BlockSpec((chunk, N), lambda i: (i, 0)),
        out_shape=jax.ShapeDtypeStruct((R, N), jnp.float32),
    )(x, wT)


def _mm_relu_add_body(x_ref, w_ref, a_ref, o_ref):
    o_ref[...] = jnp.maximum(
        a_ref[...] +
        jnp.dot(x_ref[...], w_ref[...], preferred_element_type=jnp.float32), 0.0)


def _mm_relu_add(x, wT, add, chunk):
    R, K = x.shape
    N = wT.shape[1]
    return pl.pallas_call(
        _mm_relu_add_body,
        grid=(R // chunk,),
        in_specs=[
            pl.BlockSpec((chunk, K), lambda i: (i, 0)),
            pl.BlockSpec((K, N), lambda i: (0, 0)),
            pl.BlockSpec((chunk, N), lambda i: (i, 0)),
        ],
        out_specs=pl.BlockSpec((chunk, N), lambda i: (i, 0)),
        out_shape=jax.ShapeDtypeStruct((R, N), jnp.float32),
    )(x, wT, add)


def _node_body(agg_ref, ma_ref, ia_ref, w1_ref, w2_ref, w3_ref, b_ref,
               node_ref, msg_ref, h0_ref):
    node = (jnp.dot(agg_ref[...], w1_ref[...], preferred_element_type=jnp.float32)
            + jnp.dot(ma_ref[...], w2_ref[...], preferred_element_type=jnp.float32)
            + jnp.dot(ia_ref[...], w3_ref[...], preferred_element_type=jnp.float32))
    node_ref[...] = node
    msg_ref[...] = jnp.maximum(node + b_ref[...], 0.0)
    nm = node.reshape(-1, ATOMS_PER_MOL, H)
    h0_ref[...] = nm.max(axis=1)


def _node_proj(agg, ma, ia, lrWT, bias, chunk):
    R = agg.shape[0]
    w1 = lrWT[0:H]
    w2 = lrWT[H:2 * H]
    w3 = lrWT[2 * H:3 * H]
    mols = chunk // ATOMS_PER_MOL
    return pl.pallas_call(
        _node_body,
        grid=(R // chunk,),
        in_specs=[
            pl.BlockSpec((chunk, H), lambda i: (i, 0)),
            pl.BlockSpec((chunk, H), lambda i: (i, 0)),
            pl.BlockSpec((chunk, H), lambda i: (i, 0)),
            pl.BlockSpec((H, H), lambda i: (0, 0)),
            pl.BlockSpec((H, H), lambda i: (0, 0)),
            pl.BlockSpec((H, H), lambda i: (0, 0)),
            pl.BlockSpec((1, H), lambda i: (0, 0)),
        ],
        out_specs=[
            pl.BlockSpec((chunk, H), lambda i: (i, 0)),
            pl.BlockSpec((chunk, H), lambda i: (i, 0)),
            pl.BlockSpec((mols, H), lambda i: (i, 0)),
        ],
        out_shape=[
            jax.ShapeDtypeStruct((R, H), jnp.float32),
            jax.ShapeDtypeStruct((R, H), jnp.float32),
            jax.ShapeDtypeStruct((R // ATOMS_PER_MOL, H), jnp.float32),
        ],
    )(agg, ma, ia, w1, w2, w3, bias)


def _mm_bias_body(x_ref, w_ref, b_ref, o_ref):
    o_ref[...] = (jnp.dot(x_ref[...], w_ref[...],
                          preferred_element_type=jnp.float32) + b_ref[...])


def _mm_bias(x, wT, bias, chunk):
    R, K = x.shape
    N = wT.shape[1]
    return pl.pallas_call(
        _mm_bias_body,
        grid=(R // chunk,),
        in_specs=[
            pl.BlockSpec((chunk, K), lambda i: (i, 0)),
            pl.BlockSpec((K, N), lambda i: (0, 0)),
            pl.BlockSpec((1, N), lambda i: (0, 0)),
        ],
        out_specs=pl.BlockSpec((chunk, N), lambda i: (i, 0)),
        out_shape=jax.ShapeDtypeStruct((R, N), jnp.float32),
    )(x, wT, bias)


def _gru_cell(gi, h, whh, bhh):
    gh = jnp.dot(h, whh, preferred_element_type=jnp.float32) + bhh
    r = jax.nn.sigmoid(gi[:, 0:H] + gh[:, 0:H])
    z = jax.nn.sigmoid(gi[:, H:2 * H] + gh[:, H:2 * H])
    n = jnp.tanh(gi[:, 2 * H:3 * H] + r * gh[:, 2 * H:3 * H])
    return (1.0 - z) * n + z * h


def _gru_body(gif_ref, gib_ref, h0_ref, whhf_ref, bhhf_ref,
              whhb_ref, bhhb_ref, wof_ref, wob_ref,
              out_ref, outf_s):
    T = ATOMS_PER_MOL
    B = N_MOLS

    def slab(ref, t):
        return ref[:, pl.ds(t, 1), :].reshape(B, 3 * H)

    whhf = whhf_ref[...]
    bhhf = bhhf_ref[...]
    whhb = whhb_ref[...]
    bhhb = bhhb_ref[...]
    wof = wof_ref[...]
    wob = wob_ref[...]
    h0 = h0_ref[...]

    def stepf(t, h):
        h = _gru_cell(slab(gif_ref, t), h, whhf, bhhf)
        outf_s[pl.ds(t, 1)] = h.reshape(1, B, H)
        return h

    lax.fori_loop(0, T, stepf, h0)

    def stepb(i, carry):
        h, acc = carry
        t = T - 1 - i
        h = _gru_cell(slab(gib_ref, t), h, whhb, bhhb)
        y = jnp.maximum(
            jnp.dot(outf_s[pl.ds(t, 1)].reshape(B, H), wof,
                    preferred_element_type=jnp.float32)
            + jnp.dot(h, wob, preferred_element_type=jnp.float32), 0.0)
        return h, acc + y

    _, acc = lax.fori_loop(
        0, T, stepb, (h0, jnp.zeros((B, H), jnp.float32)))
    out_ref[...] = acc * (1.0 / T)


def _gru_readout(gif_tm, gib_tm, h0, whhf, bhhf, whhb, bhhb, wof, wob):
    return pl.pallas_call(
        _gru_body,
        out_shape=jax.ShapeDtypeStruct((N_MOLS, H), jnp.float32),
        scratch_shapes=[pltpu.VMEM((ATOMS_PER_MOL, N_MOLS, H), jnp.float32)],
    )(gif_tm, gib_tm, h0, whhf, bhhf, whhb, bhhb, wof, wob)


def kernel(f_atoms, f_bonds, a2b, b2a, b2revb, n_mols, atoms_per_mol,
           W_i_atom, W_i_bond, W_h_0, W_h_1, W_o, lr_W, gru_bias,
           gru_Wih_f, gru_Whh_f, gru_bih_f, gru_bhh_f,
           gru_Wih_b, gru_Whh_b, gru_bih_b, gru_bhh_b):
    fa = jnp.pad(f_atoms, ((0, ATOM_PAD - N_ATOMS), (0, 0)))
    fb = jnp.pad(f_bonds, ((0, BOND_PAD - N_BONDS), (0, 0)))
    a2b_flat = jnp.pad(a2b.reshape(-1), (0, (ATOM_PAD - N_ATOMS) * MAX_NB))
    b2a_p = jnp.pad(b2a, (0, BOND_PAD - N_BONDS))
    b2revb_p = jnp.pad(b2revb, (0, BOND_PAD - N_BONDS))

    ia = _mm_relu(fa, W_i_atom.T, 128)
    ib = _mm_relu(fb, W_i_bond.T, 2048)

    atom_agg, bond_gather = _sc_kernels()
    ma = atom_agg(ib, a2b_flat, ia)
    t = bond_gather(ma, ib, b2a_p, b2revb_p)
    mb = _mm_relu_add(t, W_h_0.T, ib, 2048)
    ma = atom_agg(mb, a2b_flat, ma)
    t = bond_gather(ma, mb, b2a_p, b2revb_p)
    mb = _mm_relu_add(t, W_h_1.T, ib, 2048)
    agg = atom_agg(mb, a2b_flat, jnp.zeros((ATOM_PAD, H), jnp.float32))

    node, message, h0 = _node_proj(agg[1:N_ATOMS], ma[1:N_ATOMS], ia[1:N_ATOMS],
                                   lr_W.T, gru_bias.reshape(1, H), 400)

    gif = _mm_bias(message, gru_Wih_f.T, gru_bih_f.reshape(1, 3 * H), 400)
    gib = _mm_bias(message, gru_Wih_b.T, gru_bih_b.reshape(1, 3 * H), 400)
    gif_tm = gif.reshape(N_MOLS, ATOMS_PER_MOL, 3 * H)
    gib_tm = gib.reshape(N_MOLS, ATOMS_PER_MOL, 3 * H)

    mol_vecs = _gru_readout(
        gif_tm, gib_tm, h0,
        gru_Whh_f.T, gru_bhh_f.reshape(1, 3 * H),
        gru_Whh_b.T, gru_bhh_b.reshape(1, 3 * H),
        W_o[:, 0:H].T, W_o[:, H:2 * H].T)
    return mol_vecs


# speedup vs baseline: 1.1205x; 1.1205x over previous
import functools

import jax
import jax.numpy as jnp
from jax import lax
from jax.experimental import pallas as pl
from jax.experimental.pallas import tpu as pltpu
from jax.experimental.pallas import tpu_sc as plsc

H = 128
ATOM_F = 128
BOND_F = 16
N_MOLS = 200
ATOMS_PER_MOL = 50
N_ATOMS = 1 + N_MOLS * ATOMS_PER_MOL
N_BONDS = 1 + 320000
MAX_NB = 32

NW = 32
A_CHUNK = 4
PER_A = 320
ATOM_PAD = NW * PER_A
B_CHUNK = 128
PER_B = 10112
BOND_PAD = NW * PER_B


def _worker_id():
    return lax.axis_index("s") * 2 + lax.axis_index("c")


_G = A_CHUNK * MAX_NB
_NCH_A = PER_A // A_CHUNK


def _atom_agg_body(mb_hbm, a2b_hbm, ma_hbm, out_hbm,
                   idx_all, ma_all, rows_v, g0, g1, g2, g3):
    wid = _worker_id()
    ibase = pl.multiple_of(wid * (PER_A * MAX_NB), 1024)
    abase = pl.multiple_of(wid * PER_A, 64)
    pltpu.sync_copy(a2b_hbm.at[pl.ds(ibase, PER_A * MAX_NB)], idx_all)
    pltpu.sync_copy(ma_hbm.at[pl.ds(abase, PER_A)], ma_all)

    def issue(c, half, sem):
        ioff = pl.multiple_of(c * _G, _G)
        pltpu.async_copy(mb_hbm.at[idx_all.at[pl.ds(ioff, _G)]],
                         rows_v.at[pl.ds(half * _G, _G)], sem)

    gsems = (g0, g1, g2, g3)
    for p in range(3):
        issue(p, p, gsems[p])

    def chunk_body(ci, _):
        par = lax.rem(ci, 4)
        nxt = ci + 3

        for p in range(4):
            @pl.when(jnp.logical_and(nxt < _NCH_A, par == p))
            def _(p=p):
                issue(nxt, (p + 3) % 4, gsems[(p + 3) % 4])

        for p in range(4):
            @pl.when(par == p)
            def _(p=p):
                pltpu.make_async_copy(
                    mb_hbm.at[pl.ds(0, _G)],
                    rows_v.at[pl.ds(p * _G, _G)], gsems[p]).wait()

        off = par * _G
        aoff = ci * A_CHUNK
        for a in range(A_CHUNK):
            for c8 in range(H // 16):
                sl = pl.ds(c8 * 16, 16)
                vals = [rows_v[off + a * MAX_NB + j, sl] for j in range(MAX_NB)]
                m = vals[0]
                for j in range(1, MAX_NB):
                    m = jnp.maximum(m, vals[j])
                b = [((vals[s] + vals[s + 8]) + vals[s + 16]) + vals[s + 24]
                     for s in range(8)]
                c = [b[s] + b[s + 4] for s in range(4)]
                e = (c[0] + c[2]) + (c[1] + c[3])
                ma_all[aoff + a, sl] = ma_all[aoff + a, sl] + e * m
        return 0

    lax.fori_loop(0, _NCH_A, chunk_body, 0)
    pltpu.sync_copy(ma_all, out_hbm.at[pl.ds(abase, PER_A)])


_NCH_B = PER_B // B_CHUNK


def _bond_gather_body(ma_hbm, mb_hbm, b2a_hbm, b2revb_hbm, out_hbm,
                      idxa_all, idxr_all, bufa, bufr,
                      ga0, ga1, ga2, gr0, gr1, gr2, o0, o1, o2):
    wid = _worker_id()
    bbase = pl.multiple_of(wid * PER_B, 128)
    pltpu.sync_copy(b2a_hbm.at[pl.ds(bbase, PER_B)], idxa_all)
    pltpu.sync_copy(b2revb_hbm.at[pl.ds(bbase, PER_B)], idxr_all)

    gas = (ga0, ga1, ga2)
    grs = (gr0, gr1, gr2)
    os_ = (o0, o1, o2)

    def issue(c, half):
        coff = pl.multiple_of(c * B_CHUNK, B_CHUNK)
        pltpu.async_copy(ma_hbm.at[idxa_all.at[pl.ds(coff, B_CHUNK)]],
                         bufa.at[pl.ds(half * B_CHUNK, B_CHUNK)], gas[half])
        pltpu.async_copy(mb_hbm.at[idxr_all.at[pl.ds(coff, B_CHUNK)]],
                         bufr.at[pl.ds(half * B_CHUNK, B_CHUNK)], grs[half])

    def wait_g(half):
        sl = pl.ds(half * B_CHUNK, B_CHUNK)
        pltpu.make_async_copy(ma_hbm.at[pl.ds(0, B_CHUNK)], bufa.at[sl],
                              gas[half]).wait()
        pltpu.make_async_copy(mb_hbm.at[pl.ds(0, B_CHUNK)], bufr.at[sl],
                              grs[half]).wait()

    def wait_o(half):
        pltpu.make_async_copy(bufa.at[pl.ds(half * B_CHUNK, B_CHUNK)],
                              out_hbm.at[pl.ds(0, B_CHUNK)], os_[half]).wait()

    issue(0, 0)
    issue(1, 1)

    def chunk_body(ci, _):
        par = lax.rem(ci, 3)
        nxt = ci + 2

        for p in range(3):
            q = (p + 2) % 3
            cond = jnp.logical_and(nxt < _NCH_B, par == p)

            @pl.when(jnp.logical_and(cond, ci >= 1))
            def _(q=q):
                wait_o(q)

            @pl.when(cond)
            def _(q=q):
                issue(nxt, q)

        for p in range(3):
            @pl.when(par == p)
            def _(p=p):
                wait_g(p)

        off = par * B_CHUNK

        def row_body(rr, _):
            for u in range(4):
                r = off + rr * 4 + u
                for c8 in range(H // 16):
                    sl = pl.ds(c8 * 16, 16)
                    bufa[r, sl] = bufa[r, sl] - bufr[r, sl]
            return 0

        lax.fori_loop(0, B_CHUNK // 4, row_body, 0)

        base = pl.multiple_of(wid * PER_B + ci * B_CHUNK, B_CHUNK)

        for p in range(3):
            @pl.when(par == p)
            def _(p=p):
                pltpu.async_copy(bufa.at[pl.ds(p * B_CHUNK, B_CHUNK)],
                                 out_hbm.at[pl.ds(base, B_CHUNK)], os_[p])

        return 0

    lax.fori_loop(0, _NCH_B, chunk_body, 0)
    wait_o(0)
    wait_o(1)
    wait_o(2)


@functools.lru_cache(maxsize=None)
def _sc_kernels():
    mesh = plsc.VectorSubcoreMesh(core_axis_name="c", subcore_axis_name="s")
    atom_agg = pl.kernel(
        _atom_agg_body,
        out_type=jax.ShapeDtypeStruct((ATOM_PAD, H), jnp.float32),
        mesh=mesh,
        scratch_types=[
            pltpu.VMEM((PER_A * MAX_NB,), jnp.int32),
            pltpu.VMEM((PER_A, H), jnp.float32),
            pltpu.VMEM((4 * _G, H), jnp.float32),
        ] + [pltpu.SemaphoreType.DMA] * 4,
    )
    bond_gather = pl.kernel(
        _bond_gather_body,
        out_type=jax.ShapeDtypeStruct((BOND_PAD, H), jnp.float32),
        mesh=mesh,
        scratch_types=[
            pltpu.VMEM((PER_B,), jnp.int32),
            pltpu.VMEM((PER_B,), jnp.int32),
            pltpu.VMEM((3 * B_CHUNK, H), jnp.float32),
            pltpu.VMEM((3 * B_CHUNK, H), jnp.float32),
        ] + [pltpu.SemaphoreType.DMA] * 9,
    )
    return atom_agg, bond_gather


def _mm_relu_body(x_ref, w_ref, o_ref):
    o_ref[...] = jnp.maximum(
        jnp.dot(x_ref[...], w_ref[...], preferred_element_type=jnp.float32), 0.0)


def _mm_relu(x, wT, chunk):
    R, K = x.shape
    N = wT.shape[1]
    return pl.pallas_call(
        _mm_relu_body,
        grid=(R // chunk,),
        in_specs=[
            pl.BlockSpec((chunk, K), lambda i: (i, 0)),
            pl.BlockSpec((K, N), lambda i: (0, 0)),
        ],
        out_specs=pl.BlockSpec((chunk, N), lambda i: (i, 0)),
        out_shape=jax.ShapeDtypeStruct((R, N), jnp.float32),
    )(x, wT)


def _mm_relu_add_body(x_ref, w_ref, a_ref, o_ref):
    o_ref[...] = jnp.maximum(
        a_ref[...] +
        jnp.dot(x_ref[...], w_ref[...], preferred_element_type=jnp.float32), 0.0)


def _mm_relu_add(x, wT, add, chunk):
    R, K = x.shape
    N = wT.shape[1]
    return pl.pallas_call(
        _mm_relu_add_body,
        grid=(R // chunk,),
        in_specs=[
            pl.BlockSpec((chunk, K), lambda i: (i, 0)),
            pl.BlockSpec((K, N), lambda i: (0, 0)),
            pl.BlockSpec((chunk, N), lambda i: (i, 0)),
        ],
        out_specs=pl.BlockSpec((chunk, N), lambda i: (i, 0)),
        out_shape=jax.ShapeDtypeStruct((R, N), jnp.float32),
    )(x, wT, add)


def _node_body(agg_ref, ma_ref, ia_ref, w1_ref, w2_ref, w3_ref, b_ref,
               node_ref, msg_ref):
    node = (jnp.dot(agg_ref[...], w1_ref[...], preferred_element_type=jnp.float32)
            + jnp.dot(ma_ref[...], w2_ref[...], preferred_element_type=jnp.float32)
            + jnp.dot(ia_ref[...], w3_ref[...], preferred_element_type=jnp.float32))
    node_ref[...] = node
    msg_ref[...] = jnp.maximum(node + b_ref[...], 0.0)


def _node_proj(agg, ma, ia, lrWT, bias, chunk):
    R = agg.shape[0]
    w1 = lrWT[0:H]
    w2 = lrWT[H:2 * H]
    w3 = lrWT[2 * H:3 * H]
    return pl.pallas_call(
        _node_body,
        grid=(R // chunk,),
        in_specs=[
            pl.BlockSpec((chunk, H), lambda i: (i, 0)),
            pl.BlockSpec((chunk, H), lambda i: (i, 0)),
            pl.BlockSpec((chunk, H), lambda i: (i, 0)),
            pl.BlockSpec((H, H), lambda i: (0, 0)),
            pl.BlockSpec((H, H), lambda i: (0, 0)),
            pl.BlockSpec((H, H), lambda i: (0, 0)),
            pl.BlockSpec((1, H), lambda i: (0, 0)),
        ],
        out_specs=[
            pl.BlockSpec((chunk, H), lambda i: (i, 0)),
            pl.BlockSpec((chunk, H), lambda i: (i, 0)),
        ],
        out_shape=[
            jax.ShapeDtypeStruct((R, H), jnp.float32),
            jax.ShapeDtypeStruct((R, H), jnp.float32),
        ],
    )(agg, ma, ia, w1, w2, w3, bias)


def _gru_cell(x, h, wih, whh, bih, bhh):
    gi = jnp.dot(x, wih, preferred_element_type=jnp.float32) + bih
    gh = jnp.dot(h, whh, preferred_element_type=jnp.float32) + bhh
    r = jax.nn.sigmoid(gi[:, 0:H] + gh[:, 0:H])
    z = jax.nn.sigmoid(gi[:, H:2 * H] + gh[:, H:2 * H])
    n = jnp.tanh(gi[:, 2 * H:3 * H] + r * gh[:, 2 * H:3 * H])
    return (1.0 - z) * n + z * h


def _gru_body(msg_ref, hid_ref, wihf_ref, whhf_ref, bihf_ref, bhhf_ref,
              wihb_ref, whhb_ref, bihb_ref, bhhb_ref, wof_ref, wob_ref,
              out_ref, outf_s):
    T = ATOMS_PER_MOL
    B = N_MOLS

    def slab(ref, t):
        return ref[pl.ds(t, 1)].reshape(B, H)

    wihf = wihf_ref[...]
    whhf = whhf_ref[...]
    bihf = bihf_ref[...]
    bhhf = bhhf_ref[...]
    wihb = wihb_ref[...]
    whhb = whhb_ref[...]
    bihb = bihb_ref[...]
    bhhb = bhhb_ref[...]
    wof = wof_ref[...]
    wob = wob_ref[...]

    h0 = lax.fori_loop(
        1, T, lambda k, h: jnp.maximum(h, slab(hid_ref, k)), slab(hid_ref, 0))

    def stepf(t, h):
        h = _gru_cell(slab(msg_ref, t), h, wihf, whhf, bihf, bhhf)
        outf_s[pl.ds(t, 1)] = h.reshape(1, B, H)
        return h

    lax.fori_loop(0, T, stepf, h0)

    def stepb(i, carry):
        h, acc = carry
        t = T - 1 - i
        h = _gru_cell(slab(msg_ref, t), h, wihb, whhb, bihb, bhhb)
        y = jnp.maximum(
            jnp.dot(slab(outf_s, t), wof, preferred_element_type=jnp.float32)
            + jnp.dot(h, wob, preferred_element_type=jnp.float32), 0.0)
        return h, acc + y

    _, acc = lax.fori_loop(
        0, T, stepb, (h0, jnp.zeros((B, H), jnp.float32)))
    out_ref[...] = acc * (1.0 / T)


def _gru_readout(msg_tm, hid_tm, wihf, whhf, bihf, bhhf,
                 wihb, whhb, bihb, bhhb, wof, wob):
    return pl.pallas_call(
        _gru_body,
        out_shape=jax.ShapeDtypeStruct((N_MOLS, H), jnp.float32),
        scratch_shapes=[pltpu.VMEM((ATOMS_PER_MOL, N_MOLS, H), jnp.float32)],
    )(msg_tm, hid_tm, wihf, whhf, bihf, bhhf, wihb, whhb, bihb, bhhb, wof, wob)


def kernel(f_atoms, f_bonds, a2b, b2a, b2revb, n_mols, atoms_per_mol,
           W_i_atom, W_i_bond, W_h_0, W_h_1, W_o, lr_W, gru_bias,
           gru_Wih_f, gru_Whh_f, gru_bih_f, gru_bhh_f,
           gru_Wih_b, gru_Whh_b, gru_bih_b, gru_bhh_b):
    fa = jnp.pad(f_atoms, ((0, ATOM_PAD - N_ATOMS), (0, 0)))
    fb = jnp.pad(f_bonds, ((0, BOND_PAD - N_BONDS), (0, 0)))
    a2b_flat = jnp.pad(a2b.reshape(-1), (0, (ATOM_PAD - N_ATOMS) * MAX_NB))
    b2a_p = jnp.pad(b2a, (0, BOND_PAD - N_BONDS))
    b2revb_p = jnp.pad(b2revb, (0, BOND_PAD - N_BONDS))

    ia = _mm_relu(fa, W_i_atom.T, 128)
    ib = _mm_relu(fb, W_i_bond.T, 2048)

    atom_agg, bond_gather = _sc_kernels()
    ma = atom_agg(ib, a2b_flat, ia)
    t = bond_gather(ma, ib, b2a_p, b2revb_p)
    mb = _mm_relu_add(t, W_h_0.T, ib, 2048)
    ma = atom_agg(mb, a2b_flat, ma)
    t = bond_gather(ma, mb, b2a_p, b2revb_p)
    mb = _mm_relu_add(t, W_h_1.T, ib, 2048)
    agg = atom_agg(mb, a2b_flat, jnp.zeros((ATOM_PAD, H), jnp.float32))

    node, message = _node_proj(agg[1:N_ATOMS], ma[1:N_ATOMS], ia[1:N_ATOMS],
                               lr_W.T, gru_bias.reshape(1, H), 400)

    msg_tm = jnp.transpose(message.reshape(N_MOLS, ATOMS_PER_MOL, H), (1, 0, 2))
    hid_tm = jnp.transpose(node.reshape(N_MOLS, ATOMS_PER_MOL, H), (1, 0, 2))

    mol_vecs = _gru_readout(
        msg_tm, hid_tm,
        gru_Wih_f.T, gru_Whh_f.T, gru_bih_f.reshape(1, 3 * H),
        gru_bhh_f.reshape(1, 3 * H),
        gru_Wih_b.T, gru_Whh_b.T, gru_bih_b.reshape(1, 3 * H),
        gru_bhh_b.reshape(1, 3 * H),
        W_o[:, 0:H].T, W_o[:, H:2 * H].T)
    return mol_vecs


# speedup vs baseline: 1.2017x; 1.0725x over previous
import functools

import jax
import jax.numpy as jnp
from jax import lax
from jax.experimental import pallas as pl
from jax.experimental.pallas import tpu as pltpu
from jax.experimental.pallas import tpu_sc as plsc

H = 128
ATOM_F = 128
BOND_F = 16
N_MOLS = 200
ATOMS_PER_MOL = 50
N_ATOMS = 1 + N_MOLS * ATOMS_PER_MOL
N_BONDS = 1 + 320000
MAX_NB = 32

NW = 32
A_CHUNK = 4
PER_A = 320
ATOM_PAD = NW * PER_A
B_CHUNK = 128
PER_B = 10112
BOND_PAD = NW * PER_B


def _worker_id():
    return lax.axis_index("s") * 2 + lax.axis_index("c")


_G = A_CHUNK * MAX_NB
W0 = 448
W1 = 2 * PER_A - W0


def _atom_agg_body(mb_hbm, a2b_hbm, ma_hbm, out_hbm,
                   idx_all, ma_all, rows_v, g0, g1):
    c_ax = lax.axis_index("c")
    s_ax = lax.axis_index("s")
    is0 = c_ax == 0
    start = pl.multiple_of(
        s_ax * (W0 + W1) + lax.select(is0, 0, W0), 8)
    my_nch = lax.select(is0, W0 // A_CHUNK, W1 // A_CHUNK)
    ibase = pl.multiple_of(start * MAX_NB, 256)

    @pl.when(is0)
    def _():
        pltpu.sync_copy(a2b_hbm.at[pl.ds(ibase, W0 * MAX_NB)],
                        idx_all.at[pl.ds(0, W0 * MAX_NB)])
        pltpu.sync_copy(ma_hbm.at[pl.ds(start, W0)], ma_all.at[pl.ds(0, W0)])

    @pl.when(jnp.logical_not(is0))
    def _():
        pltpu.sync_copy(a2b_hbm.at[pl.ds(ibase, W1 * MAX_NB)],
                        idx_all.at[pl.ds(0, W1 * MAX_NB)])
        pltpu.sync_copy(ma_hbm.at[pl.ds(start, W1)], ma_all.at[pl.ds(0, W1)])

    def issue(c, half, sem):
        ioff = pl.multiple_of(c * _G, _G)
        pltpu.async_copy(mb_hbm.at[idx_all.at[pl.ds(ioff, _G)]],
                         rows_v.at[pl.ds(half * _G, _G)], sem)

    gsems = (g0, g1)
    issue(0, 0, g0)

    def chunk_body(ci, _):
        par = lax.rem(ci, 2)
        nxt = ci + 1

        for p in range(2):
            @pl.when(jnp.logical_and(nxt < my_nch, par == p))
            def _(p=p):
                issue(nxt, (p + 1) % 2, gsems[(p + 1) % 2])

        for p in range(2):
            @pl.when(par == p)
            def _(p=p):
                pltpu.make_async_copy(
                    mb_hbm.at[pl.ds(0, _G)],
                    rows_v.at[pl.ds(p * _G, _G)], gsems[p]).wait()

        off = par * _G
        aoff = ci * A_CHUNK
        for a in range(A_CHUNK):
            for c8 in range(H // 16):
                sl = pl.ds(c8 * 16, 16)
                vals = [rows_v[off + a * MAX_NB + j, sl] for j in range(MAX_NB)]
                m = vals[0]
                for j in range(1, MAX_NB):
                    m = jnp.maximum(m, vals[j])
                b = [((vals[s] + vals[s + 8]) + vals[s + 16]) + vals[s + 24]
                     for s in range(8)]
                c = [b[s] + b[s + 4] for s in range(4)]
                e = (c[0] + c[2]) + (c[1] + c[3])
                ma_all[aoff + a, sl] = ma_all[aoff + a, sl] + e * m
        return 0

    lax.fori_loop(0, my_nch, chunk_body, 0)

    @pl.when(is0)
    def _():
        pltpu.sync_copy(ma_all.at[pl.ds(0, W0)], out_hbm.at[pl.ds(start, W0)])

    @pl.when(jnp.logical_not(is0))
    def _():
        pltpu.sync_copy(ma_all.at[pl.ds(0, W1)], out_hbm.at[pl.ds(start, W1)])


_NCH_B = PER_B // B_CHUNK


def _bond_gather_body(ma_hbm, mb_hbm, b2a_hbm, b2revb_hbm, out_hbm,
                      idxa_all, idxr_all, bufa, bufr,
                      ga0, ga1, ga2, gr0, gr1, gr2, o0, o1, o2):
    wid = _worker_id()
    bbase = pl.multiple_of(wid * PER_B, 128)
    pltpu.sync_copy(b2a_hbm.at[pl.ds(bbase, PER_B)], idxa_all)
    pltpu.sync_copy(b2revb_hbm.at[pl.ds(bbase, PER_B)], idxr_all)

    gas = (ga0, ga1, ga2)
    grs = (gr0, gr1, gr2)
    os_ = (o0, o1, o2)

    def issue(c, half):
        coff = pl.multiple_of(c * B_CHUNK, B_CHUNK)
        pltpu.async_copy(ma_hbm.at[idxa_all.at[pl.ds(coff, B_CHUNK)]],
                         bufa.at[pl.ds(half * B_CHUNK, B_CHUNK)], gas[half])
        pltpu.async_copy(mb_hbm.at[idxr_all.at[pl.ds(coff, B_CHUNK)]],
                         bufr.at[pl.ds(half * B_CHUNK, B_CHUNK)], grs[half])

    def wait_g(half):
        sl = pl.ds(half * B_CHUNK, B_CHUNK)
        pltpu.make_async_copy(ma_hbm.at[pl.ds(0, B_CHUNK)], bufa.at[sl],
                              gas[half]).wait()
        pltpu.make_async_copy(mb_hbm.at[pl.ds(0, B_CHUNK)], bufr.at[sl],
                              grs[half]).wait()

    def wait_o(half):
        pltpu.make_async_copy(bufa.at[pl.ds(half * B_CHUNK, B_CHUNK)],
                              out_hbm.at[pl.ds(0, B_CHUNK)], os_[half]).wait()

    issue(0, 0)
    issue(1, 1)

    def chunk_body(ci, _):
        par = lax.rem(ci, 3)
        nxt = ci + 2

        for p in range(3):
            q = (p + 2) % 3
            cond = jnp.logical_and(nxt < _NCH_B, par == p)

            @pl.when(jnp.logical_and(cond, ci >= 1))
            def _(q=q):
                wait_o(q)

            @pl.when(cond)
            def _(q=q):
                issue(nxt, q)

        for p in range(3):
            @pl.when(par == p)
            def _(p=p):
                wait_g(p)

        off = par * B_CHUNK

        def row_body(rr, _):
            for u in range(4):
                r = off + rr * 4 + u
                for c8 in range(H // 16):
                    sl = pl.ds(c8 * 16, 16)
                    bufa[r, sl] = bufa[r, sl] - bufr[r, sl]
            return 0

        lax.fori_loop(0, B_CHUNK // 4, row_body, 0)

        base = pl.multiple_of(wid * PER_B + ci * B_CHUNK, B_CHUNK)

        for p in range(3):
            @pl.when(par == p)
            def _(p=p):
                pltpu.async_copy(bufa.at[pl.ds(p * B_CHUNK, B_CHUNK)],
                                 out_hbm.at[pl.ds(base, B_CHUNK)], os_[p])

        return 0

    lax.fori_loop(0, _NCH_B, chunk_body, 0)
    wait_o(0)
    wait_o(1)
    wait_o(2)


@functools.lru_cache(maxsize=None)
def _sc_kernels():
    mesh = plsc.VectorSubcoreMesh(core_axis_name="c", subcore_axis_name="s")
    atom_agg = pl.kernel(
        _atom_agg_body,
        out_type=jax.ShapeDtypeStruct((ATOM_PAD, H), jnp.float32),
        mesh=mesh,
        scratch_types=[
            pltpu.VMEM((W0 * MAX_NB,), jnp.int32),
            pltpu.VMEM((W0, H), jnp.float32),
            pltpu.VMEM((2 * _G, H), jnp.float32),
        ] + [pltpu.SemaphoreType.DMA] * 2,
    )
    bond_gather = pl.kernel(
        _bond_gather_body,
        out_type=jax.ShapeDtypeStruct((BOND_PAD, H), jnp.float32),
        mesh=mesh,
        scratch_types=[
            pltpu.VMEM((PER_B,), jnp.int32),
            pltpu.VMEM((PER_B,), jnp.int32),
            pltpu.VMEM((3 * B_CHUNK, H), jnp.float32),
            pltpu.VMEM((3 * B_CHUNK, H), jnp.float32),
        ] + [pltpu.SemaphoreType.DMA] * 9,
    )
    return atom_agg, bond_gather


def _mm_relu_body(x_ref, w_ref, o_ref):
    o_ref[...] = jnp.maximum(
        jnp.dot(x_ref[...], w_ref[...], preferred_element_type=jnp.float32), 0.0)


def _mm_relu(x, wT, chunk):
    R, K = x.shape
    N = wT.shape[1]
    return pl.pallas_call(
        _mm_relu_body,
        grid=(R // chunk,),
        in_specs=[
            pl.BlockSpec((chunk, K), lambda i: (i, 0)),
            pl.BlockSpec((K, N), lambda i: (0, 0)),
        ],
        out_specs=pl.BlockSpec((chunk, N), lambda i: (i, 0)),
        out_shape=jax.ShapeDtypeStruct((R, N), jnp.float32),
    )(x, wT)


def _mm_relu_add_body(x_ref, w_ref, a_ref, o_ref):
    o_ref[...] = jnp.maximum(
        a_ref[...] +
        jnp.dot(x_ref[...], w_ref[...], preferred_element_type=jnp.float32), 0.0)


def _mm_relu_add(x, wT, add, chunk):
    R, K = x.shape
    N = wT.shape[1]
    return pl.pallas_call(
        _mm_relu_add_body,
        grid=(R // chunk,),
        in_specs=[
            pl.BlockSpec((chunk, K), lambda i: (i, 0)),
            pl.BlockSpec((K, N), lambda i: (0, 0)),
            pl.BlockSpec((chunk, N), lambda i: (i, 0)),
        ],
        out_specs=pl.BlockSpec((chunk, N), lambda i: (i, 0)),
        out_shape=jax.ShapeDtypeStruct((R, N), jnp.float32),
    )(x, wT, add)


def _node_body(agg_ref, ma_ref, ia_ref, w1_ref, w2_ref, w3_ref, b_ref,
               node_ref, msg_ref):
    node = (jnp.dot(agg_ref[...], w1_ref[...], preferred_element_type=jnp.float32)
            + jnp.dot(ma_ref[...], w2_ref[...], preferred_element_type=jnp.float32)
            + jnp.dot(ia_ref[...], w3_ref[...], preferred_element_type=jnp.float32))
    node_ref[...] = node
    msg_ref[...] = jnp.maximum(node + b_ref[...], 0.0)


def _node_proj(agg, ma, ia, lrWT, bias, chunk):
    R = agg.shape[0]
    w1 = lrWT[0:H]
    w2 = lrWT[H:2 * H]
    w3 = lrWT[2 * H:3 * H]
    return pl.pallas_call(
        _node_body,
        grid=(R // chunk,),
        in_specs=[
            pl.BlockSpec((chunk, H), lambda i: (i, 0)),
            pl.BlockSpec((chunk, H), lambda i: (i, 0)),
            pl.BlockSpec((chunk, H), lambda i: (i, 0)),
            pl.BlockSpec((H, H), lambda i: (0, 0)),
            pl.BlockSpec((H, H), lambda i: (0, 0)),
            pl.BlockSpec((H, H), lambda i: (0, 0)),
            pl.BlockSpec((1, H), lambda i: (0, 0)),
        ],
        out_specs=[
            pl.BlockSpec((chunk, H), lambda i: (i, 0)),
            pl.BlockSpec((chunk, H), lambda i: (i, 0)),
        ],
        out_shape=[
            jax.ShapeDtypeStruct((R, H), jnp.float32),
            jax.ShapeDtypeStruct((R, H), jnp.float32),
        ],
    )(agg, ma, ia, w1, w2, w3, bias)


def _gru_cell(x, h, wih, whh, bih, bhh):
    gi = jnp.dot(x, wih, preferred_element_type=jnp.float32) + bih
    gh = jnp.dot(h, whh, preferred_element_type=jnp.float32) + bhh
    r = jax.nn.sigmoid(gi[:, 0:H] + gh[:, 0:H])
    z = jax.nn.sigmoid(gi[:, H:2 * H] + gh[:, H:2 * H])
    n = jnp.tanh(gi[:, 2 * H:3 * H] + r * gh[:, 2 * H:3 * H])
    return (1.0 - z) * n + z * h


def _gru_body(msg_ref, hid_ref, wihf_ref, whhf_ref, bihf_ref, bhhf_ref,
              wihb_ref, whhb_ref, bihb_ref, bhhb_ref, wof_ref, wob_ref,
              out_ref, outf_s):
    T = ATOMS_PER_MOL
    B = N_MOLS

    def slab(ref, t):
        return ref[pl.ds(t, 1)].reshape(B, H)

    wihf = wihf_ref[...]
    whhf = whhf_ref[...]
    bihf = bihf_ref[...]
    bhhf = bhhf_ref[...]
    wihb = wihb_ref[...]
    whhb = whhb_ref[...]
    bihb = bihb_ref[...]
    bhhb = bhhb_ref[...]
    wof = wof_ref[...]
    wob = wob_ref[...]

    h0 = lax.fori_loop(
        1, T, lambda k, h: jnp.maximum(h, slab(hid_ref, k)), slab(hid_ref, 0))

    def stepf(t, h):
        h = _gru_cell(slab(msg_ref, t), h, wihf, whhf, bihf, bhhf)
        outf_s[pl.ds(t, 1)] = h.reshape(1, B, H)
        return h

    lax.fori_loop(0, T, stepf, h0)

    def stepb(i, carry):
        h, acc = carry
        t = T - 1 - i
        h = _gru_cell(slab(msg_ref, t), h, wihb, whhb, bihb, bhhb)
        y = jnp.maximum(
            jnp.dot(slab(outf_s, t), wof, preferred_element_type=jnp.float32)
            + jnp.dot(h, wob, preferred_element_type=jnp.float32), 0.0)
        return h, acc + y

    _, acc = lax.fori_loop(
        0, T, stepb, (h0, jnp.zeros((B, H), jnp.float32)))
    out_ref[...] = acc * (1.0 / T)


def _gru_readout(msg_tm, hid_tm, wihf, whhf, bihf, bhhf,
                 wihb, whhb, bihb, bhhb, wof, wob):
    return pl.pallas_call(
        _gru_body,
        out_shape=jax.ShapeDtypeStruct((N_MOLS, H), jnp.float32),
        scratch_shapes=[pltpu.VMEM((ATOMS_PER_MOL, N_MOLS, H), jnp.float32)],
    )(msg_tm, hid_tm, wihf, whhf, bihf, bhhf, wihb, whhb, bihb, bhhb, wof, wob)


def kernel(f_atoms, f_bonds, a2b, b2a, b2revb, n_mols, atoms_per_mol,
           W_i_atom, W_i_bond, W_h_0, W_h_1, W_o, lr_W, gru_bias,
           gru_Wih_f, gru_Whh_f, gru_bih_f, gru_bhh_f,
           gru_Wih_b, gru_Whh_b, gru_bih_b, gru_bhh_b):
    fa = jnp.pad(f_atoms, ((0, ATOM_PAD - N_ATOMS), (0, 0)))
    fb = jnp.pad(f_bonds, ((0, BOND_PAD - N_BONDS), (0, 0)))
    a2b_flat = jnp.pad(a2b.reshape(-1), (0, (ATOM_PAD - N_ATOMS) * MAX_NB))
    b2a_p = jnp.pad(b2a, (0, BOND_PAD - N_BONDS))
    b2revb_p = jnp.pad(b2revb, (0, BOND_PAD - N_BONDS))

    ia = _mm_relu(fa, W_i_atom.T, 128)
    ib = _mm_relu(fb, W_i_bond.T, 2048)

    atom_agg, bond_gather = _sc_kernels()
    ma = atom_agg(ib, a2b_flat, ia)
    t = bond_gather(ma, ib, b2a_p, b2revb_p)
    mb = _mm_relu_add(t, W_h_0.T, ib, 2048)
    ma = atom_agg(mb, a2b_flat, ma)
    t = bond_gather(ma, mb, b2a_p, b2revb_p)
    mb = _mm_relu_add(t, W_h_1.T, ib, 2048)
    agg = atom_agg(mb, a2b_flat, jnp.zeros((ATOM_PAD, H), jnp.float32))

    node, message = _node_proj(agg[1:N_ATOMS], ma[1:N_ATOMS], ia[1:N_ATOMS],
                               lr_W.T, gru_bias.reshape(1, H), 400)

    msg_tm = jnp.transpose(message.reshape(N_MOLS, ATOMS_PER_MOL, H), (1, 0, 2))
    hid_tm = jnp.transpose(node.reshape(N_MOLS, ATOMS_PER_MOL, H), (1, 0, 2))

    mol_vecs = _gru_readout(
        msg_tm, hid_tm,
        gru_Wih_f.T, gru_Whh_f.T, gru_bih_f.reshape(1, 3 * H),
        gru_bhh_f.reshape(1, 3 * H),
        gru_Wih_b.T, gru_Whh_b.T, gru_bih_b.reshape(1, 3 * H),
        gru_bhh_b.reshape(1, 3 * H),
        W_o[:, 0:H].T, W_o[:, H:2 * H].T)
    return mol_vecs


# speedup vs baseline: 1.2214x; 1.0164x over previous
import functools

import jax
import jax.numpy as jnp
from jax import lax
from jax.experimental import pallas as pl
from jax.experimental.pallas import tpu as pltpu
from jax.experimental.pallas import tpu_sc as plsc

H = 128
ATOM_F = 128
BOND_F = 16
N_MOLS = 200
ATOMS_PER_MOL = 50
N_ATOMS = 1 + N_MOLS * ATOMS_PER_MOL
N_BONDS = 1 + 320000
MAX_NB = 32

NW = 32
A_CHUNK = 4
PER_A = 320
ATOM_PAD = NW * PER_A
B_CHUNK = 128
PER_B = 10112
BOND_PAD = NW * PER_B


def _worker_id():
    return lax.axis_index("s") * 2 + lax.axis_index("c")


_G = A_CHUNK * MAX_NB
W0 = 448
W1 = 2 * PER_A - W0


def _atom_agg_body(mb_hbm, a2b_hbm, ma_hbm, out_hbm,
                   idx_all, ma_all, rows_v, g0, g1):
    c_ax = lax.axis_index("c")
    s_ax = lax.axis_index("s")
    is0 = c_ax == 0
    start = pl.multiple_of(
        s_ax * (W0 + W1) + lax.select(is0, 0, W0), 8)
    my_nch = lax.select(is0, W0 // A_CHUNK, W1 // A_CHUNK)
    ibase = pl.multiple_of(start * MAX_NB, 256)

    @pl.when(is0)
    def _():
        pltpu.sync_copy(a2b_hbm.at[pl.ds(ibase, W0 * MAX_NB)],
                        idx_all.at[pl.ds(0, W0 * MAX_NB)])
        pltpu.sync_copy(ma_hbm.at[pl.ds(start, W0)], ma_all.at[pl.ds(0, W0)])

    @pl.when(jnp.logical_not(is0))
    def _():
        pltpu.sync_copy(a2b_hbm.at[pl.ds(ibase, W1 * MAX_NB)],
                        idx_all.at[pl.ds(0, W1 * MAX_NB)])
        pltpu.sync_copy(ma_hbm.at[pl.ds(start, W1)], ma_all.at[pl.ds(0, W1)])

    def issue(c, half, sem):
        ioff = pl.multiple_of(c * _G, _G)
        pltpu.async_copy(mb_hbm.at[idx_all.at[pl.ds(ioff, _G)]],
                         rows_v.at[pl.ds(half * _G, _G)], sem)

    gsems = (g0, g1)
    issue(0, 0, g0)

    def chunk_body(ci, _):
        par = lax.rem(ci, 2)
        nxt = ci + 1

        for p in range(2):
            @pl.when(jnp.logical_and(nxt < my_nch, par == p))
            def _(p=p):
                issue(nxt, (p + 1) % 2, gsems[(p + 1) % 2])

        for p in range(2):
            @pl.when(par == p)
            def _(p=p):
                pltpu.make_async_copy(
                    mb_hbm.at[pl.ds(0, _G)],
                    rows_v.at[pl.ds(p * _G, _G)], gsems[p]).wait()

        off = par * _G
        aoff = ci * A_CHUNK
        for a in range(A_CHUNK):
            for c8 in range(H // 16):
                sl = pl.ds(c8 * 16, 16)
                vals = [rows_v[off + a * MAX_NB + j, sl] for j in range(MAX_NB)]
                m = vals[0]
                for j in range(1, MAX_NB):
                    m = jnp.maximum(m, vals[j])
                b = [((vals[s] + vals[s + 8]) + vals[s + 16]) + vals[s + 24]
                     for s in range(8)]
                c = [b[s] + b[s + 4] for s in range(4)]
                e = (c[0] + c[2]) + (c[1] + c[3])
                ma_all[aoff + a, sl] = ma_all[aoff + a, sl] + e * m
        return 0

    lax.fori_loop(0, my_nch, chunk_body, 0)

    @pl.when(is0)
    def _():
        pltpu.sync_copy(ma_all.at[pl.ds(0, W0)], out_hbm.at[pl.ds(start, W0)])

    @pl.when(jnp.logical_not(is0))
    def _():
        pltpu.sync_copy(ma_all.at[pl.ds(0, W1)], out_hbm.at[pl.ds(start, W1)])


W0B = 11264
W1B = 2 * PER_B - W0B


def _bond_gather_body(ma_hbm, mb_hbm, b2a_hbm, b2revb_hbm, out_hbm,
                      idxa_all, idxr_all, bufa, bufr,
                      ga0, ga1, ga2, gr0, gr1, gr2, o0, o1, o2):
    c_ax = lax.axis_index("c")
    s_ax = lax.axis_index("s")
    is0 = c_ax == 0
    start = pl.multiple_of(
        s_ax * (W0B + W1B) + lax.select(is0, 0, W0B), B_CHUNK)
    my_nch = lax.select(is0, W0B // B_CHUNK, W1B // B_CHUNK)

    @pl.when(is0)
    def _():
        pltpu.sync_copy(b2a_hbm.at[pl.ds(start, W0B)],
                        idxa_all.at[pl.ds(0, W0B)])
        pltpu.sync_copy(b2revb_hbm.at[pl.ds(start, W0B)],
                        idxr_all.at[pl.ds(0, W0B)])

    @pl.when(jnp.logical_not(is0))
    def _():
        pltpu.sync_copy(b2a_hbm.at[pl.ds(start, W1B)],
                        idxa_all.at[pl.ds(0, W1B)])
        pltpu.sync_copy(b2revb_hbm.at[pl.ds(start, W1B)],
                        idxr_all.at[pl.ds(0, W1B)])

    gas = (ga0, ga1, ga2)
    grs = (gr0, gr1, gr2)
    os_ = (o0, o1, o2)

    def issue(c, half):
        coff = pl.multiple_of(c * B_CHUNK, B_CHUNK)
        pltpu.async_copy(ma_hbm.at[idxa_all.at[pl.ds(coff, B_CHUNK)]],
                         bufa.at[pl.ds(half * B_CHUNK, B_CHUNK)], gas[half])
        pltpu.async_copy(mb_hbm.at[idxr_all.at[pl.ds(coff, B_CHUNK)]],
                         bufr.at[pl.ds(half * B_CHUNK, B_CHUNK)], grs[half])

    def wait_g(half):
        sl = pl.ds(half * B_CHUNK, B_CHUNK)
        pltpu.make_async_copy(ma_hbm.at[pl.ds(0, B_CHUNK)], bufa.at[sl],
                              gas[half]).wait()
        pltpu.make_async_copy(mb_hbm.at[pl.ds(0, B_CHUNK)], bufr.at[sl],
                              grs[half]).wait()

    def wait_o(half):
        pltpu.make_async_copy(bufa.at[pl.ds(half * B_CHUNK, B_CHUNK)],
                              out_hbm.at[pl.ds(0, B_CHUNK)], os_[half]).wait()

    issue(0, 0)
    issue(1, 1)

    def chunk_body(ci, _):
        par = lax.rem(ci, 3)
        nxt = ci + 2

        for p in range(3):
            q = (p + 2) % 3
            cond = jnp.logical_and(nxt < my_nch, par == p)

            @pl.when(jnp.logical_and(cond, ci >= 1))
            def _(q=q):
                wait_o(q)

            @pl.when(cond)
            def _(q=q):
                issue(nxt, q)

        for p in range(3):
            @pl.when(par == p)
            def _(p=p):
                wait_g(p)

        off = par * B_CHUNK

        def row_body(rr, _):
            for u in range(4):
                r = off + rr * 4 + u
                for c8 in range(H // 16):
                    sl = pl.ds(c8 * 16, 16)
                    bufa[r, sl] = bufa[r, sl] - bufr[r, sl]
            return 0

        lax.fori_loop(0, B_CHUNK // 4, row_body, 0)

        base = pl.multiple_of(start + ci * B_CHUNK, B_CHUNK)

        for p in range(3):
            @pl.when(par == p)
            def _(p=p):
                pltpu.async_copy(bufa.at[pl.ds(p * B_CHUNK, B_CHUNK)],
                                 out_hbm.at[pl.ds(base, B_CHUNK)], os_[p])

        return 0

    lax.fori_loop(0, my_nch, chunk_body, 0)
    wait_o(0)
    wait_o(1)
    wait_o(2)


@functools.lru_cache(maxsize=None)
def _sc_kernels():
    mesh = plsc.VectorSubcoreMesh(core_axis_name="c", subcore_axis_name="s")
    atom_agg = pl.kernel(
        _atom_agg_body,
        out_type=jax.ShapeDtypeStruct((ATOM_PAD, H), jnp.float32),
        mesh=mesh,
        scratch_types=[
            pltpu.VMEM((W0 * MAX_NB,), jnp.int32),
            pltpu.VMEM((W0, H), jnp.float32),
            pltpu.VMEM((2 * _G, H), jnp.float32),
        ] + [pltpu.SemaphoreType.DMA] * 2,
    )
    bond_gather = pl.kernel(
        _bond_gather_body,
        out_type=jax.ShapeDtypeStruct((BOND_PAD, H), jnp.float32),
        mesh=mesh,
        scratch_types=[
            pltpu.VMEM((W0B,), jnp.int32),
            pltpu.VMEM((W0B,), jnp.int32),
            pltpu.VMEM((3 * B_CHUNK, H), jnp.float32),
            pltpu.VMEM((3 * B_CHUNK, H), jnp.float32),
        ] + [pltpu.SemaphoreType.DMA] * 9,
    )
    return atom_agg, bond_gather


def _mm_relu_body(x_ref, w_ref, o_ref):
    o_ref[...] = jnp.maximum(
        jnp.dot(x_ref[...], w_ref[...], preferred_element_type=jnp.float32), 0.0)


def _mm_relu(x, wT, chunk):
    R, K = x.shape
    N = wT.shape[1]
    return pl.pallas_call(
        _mm_relu_body,
        grid=(R // chunk,),
        in_specs=[
            pl.BlockSpec((chunk, K), lambda i: (i, 0)),
            pl.BlockSpec((K, N), lambda i: (0, 0)),
        ],
        out_specs=pl.BlockSpec((chunk, N), lambda i: (i, 0)),
        out_shape=jax.ShapeDtypeStruct((R, N), jnp.float32),
    )(x, wT)


def _mm_relu_add_body(x_ref, w_ref, a_ref, o_ref):
    o_ref[...] = jnp.maximum(
        a_ref[...] +
        jnp.dot(x_ref[...], w_ref[...], preferred_element_type=jnp.float32), 0.0)


def _mm_relu_add(x, wT, add, chunk):
    R, K = x.shape
    N = wT.shape[1]
    return pl.pallas_call(
        _mm_relu_add_body,
        grid=(R // chunk,),
        in_specs=[
            pl.BlockSpec((chunk, K), lambda i: (i, 0)),
            pl.BlockSpec((K, N), lambda i: (0, 0)),
            pl.BlockSpec((chunk, N), lambda i: (i, 0)),
        ],
        out_specs=pl.BlockSpec((chunk, N), lambda i: (i, 0)),
        out_shape=jax.ShapeDtypeStruct((R, N), jnp.float32),
    )(x, wT, add)


def _node_body(agg_ref, ma_ref, ia_ref, w1_ref, w2_ref, w3_ref, b_ref,
               node_ref, msg_ref):
    node = (jnp.dot(agg_ref[...], w1_ref[...], preferred_element_type=jnp.float32)
            + jnp.dot(ma_ref[...], w2_ref[...], preferred_element_type=jnp.float32)
            + jnp.dot(ia_ref[...], w3_ref[...], preferred_element_type=jnp.float32))
    node_ref[...] = node
    msg_ref[...] = jnp.maximum(node + b_ref[...], 0.0)


def _node_proj(agg, ma, ia, lrWT, bias, chunk):
    R = agg.shape[0]
    w1 = lrWT[0:H]
    w2 = lrWT[H:2 * H]
    w3 = lrWT[2 * H:3 * H]
    return pl.pallas_call(
        _node_body,
        grid=(R // chunk,),
        in_specs=[
            pl.BlockSpec((chunk, H), lambda i: (i, 0)),
            pl.BlockSpec((chunk, H), lambda i: (i, 0)),
            pl.BlockSpec((chunk, H), lambda i: (i, 0)),
            pl.BlockSpec((H, H), lambda i: (0, 0)),
            pl.BlockSpec((H, H), lambda i: (0, 0)),
            pl.BlockSpec((H, H), lambda i: (0, 0)),
            pl.BlockSpec((1, H), lambda i: (0, 0)),
        ],
        out_specs=[
            pl.BlockSpec((chunk, H), lambda i: (i, 0)),
            pl.BlockSpec((chunk, H), lambda i: (i, 0)),
        ],
        out_shape=[
            jax.ShapeDtypeStruct((R, H), jnp.float32),
            jax.ShapeDtypeStruct((R, H), jnp.float32),
        ],
    )(agg, ma, ia, w1, w2, w3, bias)


def _gru_cell(x, h, wih, whh, bih, bhh):
    gi = jnp.dot(x, wih, preferred_element_type=jnp.float32) + bih
    gh = jnp.dot(h, whh, preferred_element_type=jnp.float32) + bhh
    r = jax.nn.sigmoid(gi[:, 0:H] + gh[:, 0:H])
    z = jax.nn.sigmoid(gi[:, H:2 * H] + gh[:, H:2 * H])
    n = jnp.tanh(gi[:, 2 * H:3 * H] + r * gh[:, 2 * H:3 * H])
    return (1.0 - z) * n + z * h


def _gru_body(msg_ref, hid_ref, wihf_ref, whhf_ref, bihf_ref, bhhf_ref,
              wihb_ref, whhb_ref, bihb_ref, bhhb_ref, wof_ref, wob_ref,
              out_ref, outf_s):
    T = ATOMS_PER_MOL
    B = N_MOLS

    def slab(ref, t):
        return ref[pl.ds(t, 1)].reshape(B, H)

    wihf = wihf_ref[...]
    whhf = whhf_ref[...]
    bihf = bihf_ref[...]
    bhhf = bhhf_ref[...]
    wihb = wihb_ref[...]
    whhb = whhb_ref[...]
    bihb = bihb_ref[...]
    bhhb = bhhb_ref[...]
    wof = wof_ref[...]
    wob = wob_ref[...]

    h0 = lax.fori_loop(
        1, T, lambda k, h: jnp.maximum(h, slab(hid_ref, k)), slab(hid_ref, 0))

    def stepf(t, h):
        h = _gru_cell(slab(msg_ref, t), h, wihf, whhf, bihf, bhhf)
        outf_s[pl.ds(t, 1)] = h.reshape(1, B, H)
        return h

    lax.fori_loop(0, T, stepf, h0)

    def stepb(i, carry):
        h, acc = carry
        t = T - 1 - i
        h = _gru_cell(slab(msg_ref, t), h, wihb, whhb, bihb, bhhb)
        y = jnp.maximum(
            jnp.dot(slab(outf_s, t), wof, preferred_element_type=jnp.float32)
            + jnp.dot(h, wob, preferred_element_type=jnp.float32), 0.0)
        return h, acc + y

    _, acc = lax.fori_loop(
        0, T, stepb, (h0, jnp.zeros((B, H), jnp.float32)))
    out_ref[...] = acc * (1.0 / T)


def _gru_readout(msg_tm, hid_tm, wihf, whhf, bihf, bhhf,
                 wihb, whhb, bihb, bhhb, wof, wob):
    return pl.pallas_call(
        _gru_body,
        out_shape=jax.ShapeDtypeStruct((N_MOLS, H), jnp.float32),
        scratch_shapes=[pltpu.VMEM((ATOMS_PER_MOL, N_MOLS, H), jnp.float32)],
    )(msg_tm, hid_tm, wihf, whhf, bihf, bhhf, wihb, whhb, bihb, bhhb, wof, wob)


def kernel(f_atoms, f_bonds, a2b, b2a, b2revb, n_mols, atoms_per_mol,
           W_i_atom, W_i_bond, W_h_0, W_h_1, W_o, lr_W, gru_bias,
           gru_Wih_f, gru_Whh_f, gru_bih_f, gru_bhh_f,
           gru_Wih_b, gru_Whh_b, gru_bih_b, gru_bhh_b):
    fa = jnp.pad(f_atoms, ((0, ATOM_PAD - N_ATOMS), (0, 0)))
    fb = jnp.pad(f_bonds, ((0, BOND_PAD - N_BONDS), (0, 0)))
    a2b_flat = jnp.pad(a2b.reshape(-1), (0, (ATOM_PAD - N_ATOMS) * MAX_NB))
    b2a_p = jnp.pad(b2a, (0, BOND_PAD - N_BONDS))
    b2revb_p = jnp.pad(b2revb, (0, BOND_PAD - N_BONDS))

    ia = _mm_relu(fa, W_i_atom.T, 128)
    ib = _mm_relu(fb, W_i_bond.T, 2048)

    atom_agg, bond_gather = _sc_kernels()
    ma = atom_agg(ib, a2b_flat, ia)
    t = bond_gather(ma, ib, b2a_p, b2revb_p)
    mb = _mm_relu_add(t, W_h_0.T, ib, 2048)
    ma = atom_agg(mb, a2b_flat, ma)
    t = bond_gather(ma, mb, b2a_p, b2revb_p)
    mb = _mm_relu_add(t, W_h_1.T, ib, 2048)
    agg = atom_agg(mb, a2b_flat, jnp.zeros((ATOM_PAD, H), jnp.float32))

    node, message = _node_proj(agg[1:N_ATOMS], ma[1:N_ATOMS], ia[1:N_ATOMS],
                               lr_W.T, gru_bias.reshape(1, H), 400)

    msg_tm = jnp.transpose(message.reshape(N_MOLS, ATOMS_PER_MOL, H), (1, 0, 2))
    hid_tm = jnp.transpose(node.reshape(N_MOLS, ATOMS_PER_MOL, H), (1, 0, 2))

    mol_vecs = _gru_readout(
        msg_tm, hid_tm,
        gru_Wih_f.T, gru_Whh_f.T, gru_bih_f.reshape(1, 3 * H),
        gru_bhh_f.reshape(1, 3 * H),
        gru_Wih_b.T, gru_Whh_b.T, gru_bih_b.reshape(1, 3 * H),
        gru_bhh_b.reshape(1, 3 * H),
        W_o[:, 0:H].T, W_o[:, H:2 * H].T)
    return mol_vecs
